# baseline (speedup 1.0000x reference)
import jax
import jax.numpy as jnp
from jax import lax
from jax.experimental import pallas as pl
from jax.experimental.pallas import tpu as pltpu
from jax.experimental.pallas import tpu_sc as plsc

NEG_SLOPE = 0.01

_ROWS = 32
_NSUB = 16
_EDGE_BLK = 8192


def _build_counts(e1, e2, n, interpret=False):
    e = e1.shape[1]
    chunks = n // (_NSUB * _ROWS)
    eb_sz = min(e, _EDGE_BLK)
    n_eb = e // eb_sz
    unroll = 4

    def body(e1_hbm, e2_hbm, out_hbm, ebuf0, ebuf1, acc, sem0, sem1):
        g = lax.axis_index("c")
        s = lax.axis_index("s")

        def pipeline(ehbm):
            bufs = (ebuf0, ebuf1)
            sems = (sem0, sem1)
            for chunk in range(chunks):
                lo = (s * chunks + chunk) * _ROWS

                zf = jnp.zeros((16,), jnp.float32)

                def zbody(i, carry):
                    r = i // (n // 128)
                    c = (i % (n // 128)) * 128
                    for u in range(8):
                        acc[r, pl.ds(c + u * 16, 16)] = zf
                    return carry

                lax.fori_loop(0, (_ROWS * n) // 128, zbody, 0)

                lov = jnp.full((16,), lo, jnp.int32)
                rows_v = jnp.full((16,), _ROWS, jnp.int32)
                n_v = jnp.full((16,), n, jnp.int32)
                zero_i = jnp.zeros((16,), jnp.int32)
                one_f = jnp.ones((16,), jnp.float32)
                zero_f = jnp.zeros((16,), jnp.float32)

                cps = {0: pltpu.async_copy(
                    ehbm.at[:, pl.ds(0, eb_sz)], bufs[0], sems[0])}
                for b in range(n_eb):
                    if b + 1 < n_eb:
                        cps[b + 1] = pltpu.async_copy(
                            ehbm.at[:, pl.ds((b + 1) * eb_sz, eb_sz)],
                            bufs[(b + 1) % 2], sems[(b + 1) % 2])
                    cps[b].wait()
                    buf = bufs[b % 2]

                    def ebody(i, carry):
                        for u in range(unroll):
                            off = (i * unroll + u) * 16
                            s16 = buf[0, pl.ds(off, 16)]
                            d16 = buf[1, pl.ds(off, 16)]
                            r16 = d16 - lov
                            m = (r16 >= zero_i) & (r16 < rows_v)
                            rsafe = jnp.where(m, r16, zero_i)
                            csafe = jnp.where(m, s16, zero_i)
                            plsc.addupdate_scatter(
                                acc, [rsafe, csafe], jnp.where(m, one_f, zero_f),
                                mask=m)
                        return carry

                    lax.fori_loop(0, eb_sz // (16 * unroll), ebody, 0)

                pltpu.sync_copy(acc, out_hbm.at[g, pl.ds(lo, _ROWS), :])

        @pl.when(g == 0)
        def _():
            pipeline(e1_hbm)

        @pl.when(g == 1)
        def _():
            pipeline(e2_hbm)

    mesh = plsc.VectorSubcoreMesh(
        core_axis_name="c", subcore_axis_name="s",
        num_cores=2, num_subcores=_NSUB)
    f = pl.kernel(
        body,
        out_type=jax.ShapeDtypeStruct((2, n, n), jnp.float32),
        mesh=mesh,
        scratch_types=[
            pltpu.VMEM((2, eb_sz), jnp.int32),
            pltpu.VMEM((2, eb_sz), jnp.int32),
            pltpu.VMEM((_ROWS, n), jnp.float32),
            pltpu.SemaphoreType.DMA,
            pltpu.SemaphoreType.DMA,
        ],
        compiler_params=pltpu.CompilerParams(needs_layout_passes=False),
        interpret=interpret,
    )
    return f(e1, e2)


def _matmul_plain(x, w, interpret=False):
    n = x.shape[0]
    bm = 256

    def body(x_ref, w_ref, o_ref):
        o_ref[...] = jnp.dot(x_ref[...], w_ref[...],
                             preferred_element_type=jnp.float32
                             ).astype(jnp.bfloat16)

    return pl.pallas_call(
        body,
        grid=(n // bm,),
        in_specs=[
            pl.BlockSpec((bm, n), lambda i: (i, 0)),
            pl.BlockSpec((n, n), lambda i: (0, 0)),
        ],
        out_specs=pl.BlockSpec((bm, n), lambda i: (i, 0)),
        out_shape=jax.ShapeDtypeStruct((n, n), jnp.bfloat16),
        interpret=interpret,
    )(x, w)


def _scale_rows(t, dinvb, interpret=False):
    n = t.shape[0]
    g2 = dinvb.shape[0]
    bm = 256

    def body(t_ref, dv_ref, o_ref):
        o_ref[0] = t_ref[...] * dv_ref[0, :, 0:1]

    return pl.pallas_call(
        body,
        grid=(g2, n // bm),
        in_specs=[
            pl.BlockSpec((bm, n), lambda g, i: (i, 0)),
            pl.BlockSpec((1, bm, 128), lambda g, i: (g, i, 0)),
        ],
        out_specs=pl.BlockSpec((1, bm, n), lambda g, i: (g, i, 0)),
        out_shape=jax.ShapeDtypeStruct((g2, n, n), jnp.float32),
        interpret=interpret,
    )(t, dinvb)


def _rsqrt_deg(a3, w, interpret=False):
    g2, n, _ = a3.shape
    bm = 256

    def body(a_ref, w_ref, o_ref, ab_ref, wb_ref):
        a = a_ref[0]
        deg = jnp.sum(a, axis=1, keepdims=True) + 1.0
        o_ref[0] = jnp.broadcast_to(lax.rsqrt(deg), (a.shape[0], 128))
        ab_ref[0] = a.astype(jnp.bfloat16)
        wb_ref[...] = w_ref[...].astype(jnp.bfloat16)

    return pl.pallas_call(
        body,
        grid=(g2, n // bm),
        in_specs=[
            pl.BlockSpec((1, bm, n), lambda g, i: (g, i, 0)),
            pl.BlockSpec((bm, n), lambda g, i: (i, 0)),
        ],
        out_specs=[
            pl.BlockSpec((1, bm, 128), lambda g, i: (g, i, 0)),
            pl.BlockSpec((1, bm, n), lambda g, i: (g, i, 0)),
            pl.BlockSpec((bm, n), lambda g, i: (i, 0)),
        ],
        out_shape=[
            jax.ShapeDtypeStruct((g2, n, 128), jnp.float32),
            jax.ShapeDtypeStruct((g2, n, n), jnp.bfloat16),
            jax.ShapeDtypeStruct((n, n), jnp.bfloat16),
        ],
        interpret=interpret,
    )(a3, w)


def _layer(x, w, ab, dinvb, act, first, out_dtype=jnp.bfloat16,
           interpret=False):
    g2, n, _ = ab.shape
    bm = 256
    nb = n // bm

    def body(*refs):
        if first:
            x_ref, a_ref, dv_ref, o_ref, u_ref = refs
        else:
            x_ref, w_ref, a_ref, dv_ref, o_ref, u_ref = refs
        ph = pl.program_id(1)
        i = pl.program_id(2)
        dv = dv_ref[0, :, 0:1]

        @pl.when(ph == 0)
        def _():
            if first:
                t = x_ref[...].astype(jnp.float32)
            else:
                t = jnp.dot(x_ref[0], w_ref[...],
                            preferred_element_type=jnp.float32)
            u_ref[pl.ds(i * bm, bm), :] = (t * dv).astype(jnp.bfloat16)

        @pl.when(ph == 1)
        def _():
            y = jnp.dot(a_ref[0], u_ref[...],
                        preferred_element_type=jnp.float32)
            ue = u_ref[pl.ds(i * bm, bm), :].astype(jnp.float32)
            y = (y + ue) * dv
            if act:
                y = jnp.where(y >= 0, y, NEG_SLOPE * y)
            o_ref[0] = y.astype(out_dtype)

    if first:
        x_spec = pl.BlockSpec(
            (bm, n), lambda g, p, i: ((1 - p) * i + p * (nb - 1), 0))
        in_specs = [x_spec]
        args = (x,)
    else:
        x_spec = pl.BlockSpec(
            (1, bm, n), lambda g, p, i: (g, (1 - p) * i + p * (nb - 1), 0))
        w_spec = pl.BlockSpec((n, n), lambda g, p, i: (0, 0))
        in_specs = [x_spec, w_spec]
        args = (x, w)
    in_specs += [
        pl.BlockSpec((1, bm, n), lambda g, p, i: (g, p * i, 0)),
        pl.BlockSpec((1, bm, 128), lambda g, p, i: (g, i, 0)),
    ]
    args += (ab, dinvb)

    return pl.pallas_call(
        body,
        grid=(g2, 2, nb),
        in_specs=in_specs,
        out_specs=pl.BlockSpec((1, bm, n), lambda g, p, i: (g, p * i, 0)),
        out_shape=jax.ShapeDtypeStruct((g2, n, n), out_dtype),
        scratch_shapes=[pltpu.VMEM((n, n), jnp.bfloat16)],
        interpret=interpret,
    )(*args)


def _xw_scaled(x, w, dinvb, interpret=False):
    gx, n, _ = x.shape
    g2 = dinvb.shape[0]
    bm = 256

    def body(x_ref, w_ref, dv_ref, o_ref):
        xb = x_ref[0]
        dv = dv_ref[0, :, 0:1]
        o_ref[0] = jnp.dot(xb, w_ref[...], preferred_element_type=jnp.float32) * dv

    xmap = (lambda g, i: (g, i, 0)) if gx == g2 else (lambda g, i: (0, i, 0))
    return pl.pallas_call(
        body,
        grid=(g2, n // bm),
        in_specs=[
            pl.BlockSpec((1, bm, n), xmap),
            pl.BlockSpec((n, n), lambda g, i: (0, 0)),
            pl.BlockSpec((1, bm, 128), lambda g, i: (g, i, 0)),
        ],
        out_specs=pl.BlockSpec((1, bm, n), lambda g, i: (g, i, 0)),
        out_shape=jax.ShapeDtypeStruct((g2, n, n), jnp.float32),
        interpret=interpret,
    )(x, w, dinvb)


def _spread(ab, u, dinvb, act, interpret=False):
    g2, n, _ = ab.shape
    bm = 256

    def body(a_ref, uf_ref, dv_ref, o_ref):
        i = pl.program_id(1)
        a = a_ref[0].astype(jnp.float32)
        y = jnp.dot(a, uf_ref[0], preferred_element_type=jnp.float32)
        ue = uf_ref[0, pl.ds(i * bm, bm), :]
        y = (y + ue) * dv_ref[0, :, 0:1]
        if act:
            y = jnp.where(y >= 0, y, NEG_SLOPE * y)
        o_ref[0] = y

    return pl.pallas_call(
        body,
        grid=(g2, n // bm),
        in_specs=[
            pl.BlockSpec((1, bm, n), lambda g, i: (g, i, 0)),
            pl.BlockSpec((1, n, n), lambda g, i: (g, 0, 0)),
            pl.BlockSpec((1, bm, 128), lambda g, i: (g, i, 0)),
        ],
        out_specs=pl.BlockSpec((1, bm, n), lambda g, i: (g, i, 0)),
        out_shape=jax.ShapeDtypeStruct((g2, n, n), jnp.float32),
        interpret=interpret,
    )(ab, u, dinvb)


def _cross(x, interpret=False):
    _, n, _ = x.shape
    bm = 512

    def body(x1_ref, x2_ref, o_ref):
        o_ref[...] = lax.dot_general(
            x1_ref[0], x2_ref[0], (((0,), (0,)), ((), ())),
            preferred_element_type=jnp.float32)

    return pl.pallas_call(
        body,
        grid=(n // bm,),
        in_specs=[
            pl.BlockSpec((1, n, bm), lambda i: (0, 0, i)),
            pl.BlockSpec((1, n, n), lambda i: (1, 0, 0)),
        ],
        out_specs=pl.BlockSpec((bm, n), lambda i: (i, 0)),
        out_shape=jax.ShapeDtypeStruct((n, n), jnp.float32),
        interpret=interpret,
    )(x, x)


def kernel(edge_index_1, edge_index_2, feature, W):
    n = feature.shape[0]
    t1 = _matmul_plain(feature, W)
    a3 = _build_counts(edge_index_1, edge_index_2, n)
    dinvb, ab, wb = _rsqrt_deg(a3, W)
    x = _layer(t1, None, ab, dinvb, act=True, first=True)
    for layer in range(1, 4):
        x = _layer(x, wb, ab, dinvb, act=True, first=False)
    x = _layer(x, wb, ab, dinvb, act=False, first=False,
               out_dtype=jnp.float32)
    pred = _cross(x)
    return x[0], x[1], pred


# speedup vs baseline: 1.0340x; 1.0340x over previous
import jax
import jax.numpy as jnp
from jax import lax
from jax.experimental import pallas as pl
from jax.experimental.pallas import tpu as pltpu
from jax.experimental.pallas import tpu_sc as plsc

NEG_SLOPE = 0.01

_ROWS = 32
_NSUB = 16
_EDGE_BLK = 8192


def _build_counts(e1, e2, n, interpret=False):
    e = e1.shape[1]
    chunks = n // (_NSUB * _ROWS)
    eb_sz = min(e, _EDGE_BLK)
    n_eb = e // eb_sz
    unroll = 8

    def body(e1_hbm, e2_hbm, out_hbm, ebuf0, ebuf1, acc, sem0, sem1):
        g = lax.axis_index("c")
        s = lax.axis_index("s")

        def pipeline(ehbm):
            bufs = (ebuf0, ebuf1)
            sems = (sem0, sem1)
            for chunk in range(chunks):
                lo = (s * chunks + chunk) * _ROWS

                cps = {0: pltpu.async_copy(
                    ehbm.at[:, pl.ds(0, eb_sz)], bufs[0], sems[0])}

                zf = jnp.zeros((16,), jnp.float32)

                def zbody(i, carry):
                    r = i // (n // 128)
                    c = (i % (n // 128)) * 128
                    for u in range(8):
                        acc[r, pl.ds(c + u * 16, 16)] = zf
                    return carry

                lax.fori_loop(0, (_ROWS * n) // 128, zbody, 0)

                lov = jnp.full((16,), lo, jnp.int32)
                rows_u = jnp.full((16,), _ROWS, jnp.uint32)

                for b in range(n_eb):
                    if b + 1 < n_eb:
                        cps[b + 1] = pltpu.async_copy(
                            ehbm.at[:, pl.ds((b + 1) * eb_sz, eb_sz)],
                            bufs[(b + 1) % 2], sems[(b + 1) % 2])
                    cps[b].wait()
                    buf = bufs[b % 2]

                    def ebody(i, carry):
                        for u in range(unroll):
                            off = (i * unroll + u) * 16
                            s16 = buf[0, pl.ds(off, 16)]
                            d16 = buf[1, pl.ds(off, 16)]
                            r16 = d16 - lov
                            m = plsc.bitcast(r16, jnp.uint32) < rows_u
                            plsc.addupdate_scatter(
                                acc, [r16, s16], m.astype(jnp.float32),
                                mask=m)
                        return carry

                    lax.fori_loop(0, eb_sz // (16 * unroll), ebody, 0)

                pltpu.sync_copy(acc, out_hbm.at[g, pl.ds(lo, _ROWS), :])

        @pl.when(g == 0)
        def _():
            pipeline(e1_hbm)

        @pl.when(g == 1)
        def _():
            pipeline(e2_hbm)

    mesh = plsc.VectorSubcoreMesh(
        core_axis_name="c", subcore_axis_name="s",
        num_cores=2, num_subcores=_NSUB)
    f = pl.kernel(
        body,
        out_type=jax.ShapeDtypeStruct((2, n, n), jnp.float32),
        mesh=mesh,
        scratch_types=[
            pltpu.VMEM((2, eb_sz), jnp.int32),
            pltpu.VMEM((2, eb_sz), jnp.int32),
            pltpu.VMEM((_ROWS, n), jnp.float32),
            pltpu.SemaphoreType.DMA,
            pltpu.SemaphoreType.DMA,
        ],
        compiler_params=pltpu.CompilerParams(needs_layout_passes=False),
        interpret=interpret,
    )
    return f(e1, e2)


def _matmul_plain(x, w, interpret=False):
    n = x.shape[0]
    bm = 256

    def body(x_ref, w_ref, o_ref):
        o_ref[...] = jnp.dot(x_ref[...], w_ref[...],
                             preferred_element_type=jnp.float32
                             ).astype(jnp.bfloat16)

    return pl.pallas_call(
        body,
        grid=(n // bm,),
        in_specs=[
            pl.BlockSpec((bm, n), lambda i: (i, 0)),
            pl.BlockSpec((n, n), lambda i: (0, 0)),
        ],
        out_specs=pl.BlockSpec((bm, n), lambda i: (i, 0)),
        out_shape=jax.ShapeDtypeStruct((n, n), jnp.bfloat16),
        interpret=interpret,
    )(x, w)


def _scale_rows(t, dinvb, interpret=False):
    n = t.shape[0]
    g2 = dinvb.shape[0]
    bm = 256

    def body(t_ref, dv_ref, o_ref):
        o_ref[0] = t_ref[...] * dv_ref[0, :, 0:1]

    return pl.pallas_call(
        body,
        grid=(g2, n // bm),
        in_specs=[
            pl.BlockSpec((bm, n), lambda g, i: (i, 0)),
            pl.BlockSpec((1, bm, 128), lambda g, i: (g, i, 0)),
        ],
        out_specs=pl.BlockSpec((1, bm, n), lambda g, i: (g, i, 0)),
        out_shape=jax.ShapeDtypeStruct((g2, n, n), jnp.float32),
        interpret=interpret,
    )(t, dinvb)


def _rsqrt_deg(a3, w, interpret=False):
    g2, n, _ = a3.shape
    bm = 256

    def body(a_ref, w_ref, o_ref, ab_ref, wb_ref):
        a = a_ref[0]
        deg = jnp.sum(a, axis=1, keepdims=True) + 1.0
        o_ref[0] = jnp.broadcast_to(lax.rsqrt(deg), (a.shape[0], 128))
        ab_ref[0] = a.astype(jnp.bfloat16)
        wb_ref[...] = w_ref[...].astype(jnp.bfloat16)

    return pl.pallas_call(
        body,
        grid=(g2, n // bm),
        in_specs=[
            pl.BlockSpec((1, bm, n), lambda g, i: (g, i, 0)),
            pl.BlockSpec((bm, n), lambda g, i: (i, 0)),
        ],
        out_specs=[
            pl.BlockSpec((1, bm, 128), lambda g, i: (g, i, 0)),
            pl.BlockSpec((1, bm, n), lambda g, i: (g, i, 0)),
            pl.BlockSpec((bm, n), lambda g, i: (i, 0)),
        ],
        out_shape=[
            jax.ShapeDtypeStruct((g2, n, 128), jnp.float32),
            jax.ShapeDtypeStruct((g2, n, n), jnp.bfloat16),
            jax.ShapeDtypeStruct((n, n), jnp.bfloat16),
        ],
        interpret=interpret,
    )(a3, w)


def _layer(x, w, ab, dinvb, act, first, out_dtype=jnp.bfloat16,
           interpret=False):
    g2, n, _ = ab.shape
    bm = 256
    nb = n // bm

    def body(*refs):
        if first:
            x_ref, a_ref, dv_ref, o_ref, u_ref = refs
        else:
            x_ref, w_ref, a_ref, dv_ref, o_ref, u_ref = refs
        ph = pl.program_id(1)
        i = pl.program_id(2)
        dv = dv_ref[0, :, 0:1]

        @pl.when(ph == 0)
        def _():
            if first:
                t = x_ref[...].astype(jnp.float32)
            else:
                t = jnp.dot(x_ref[0], w_ref[...],
                            preferred_element_type=jnp.float32)
            u_ref[pl.ds(i * bm, bm), :] = (t * dv).astype(jnp.bfloat16)

        @pl.when(ph == 1)
        def _():
            y = jnp.dot(a_ref[0], u_ref[...],
                        preferred_element_type=jnp.float32)
            ue = u_ref[pl.ds(i * bm, bm), :].astype(jnp.float32)
            y = (y + ue) * dv
            if act:
                y = jnp.where(y >= 0, y, NEG_SLOPE * y)
            o_ref[0] = y.astype(out_dtype)

    if first:
        x_spec = pl.BlockSpec(
            (bm, n), lambda g, p, i: ((1 - p) * i + p * (nb - 1), 0))
        in_specs = [x_spec]
        args = (x,)
    else:
        x_spec = pl.BlockSpec(
            (1, bm, n), lambda g, p, i: (g, (1 - p) * i + p * (nb - 1), 0))
        w_spec = pl.BlockSpec((n, n), lambda g, p, i: (0, 0))
        in_specs = [x_spec, w_spec]
        args = (x, w)
    in_specs += [
        pl.BlockSpec((1, bm, n), lambda g, p, i: (g, p * i, 0)),
        pl.BlockSpec((1, bm, 128), lambda g, p, i: (g, i, 0)),
    ]
    args += (ab, dinvb)

    return pl.pallas_call(
        body,
        grid=(g2, 2, nb),
        in_specs=in_specs,
        out_specs=pl.BlockSpec((1, bm, n), lambda g, p, i: (g, p * i, 0)),
        out_shape=jax.ShapeDtypeStruct((g2, n, n), out_dtype),
        scratch_shapes=[pltpu.VMEM((n, n), jnp.bfloat16)],
        interpret=interpret,
    )(*args)


def _xw_scaled(x, w, dinvb, interpret=False):
    gx, n, _ = x.shape
    g2 = dinvb.shape[0]
    bm = 256

    def body(x_ref, w_ref, dv_ref, o_ref):
        xb = x_ref[0]
        dv = dv_ref[0, :, 0:1]
        o_ref[0] = jnp.dot(xb, w_ref[...], preferred_element_type=jnp.float32) * dv

    xmap = (lambda g, i: (g, i, 0)) if gx == g2 else (lambda g, i: (0, i, 0))
    return pl.pallas_call(
        body,
        grid=(g2, n // bm),
        in_specs=[
            pl.BlockSpec((1, bm, n), xmap),
            pl.BlockSpec((n, n), lambda g, i: (0, 0)),
            pl.BlockSpec((1, bm, 128), lambda g, i: (g, i, 0)),
        ],
        out_specs=pl.BlockSpec((1, bm, n), lambda g, i: (g, i, 0)),
        out_shape=jax.ShapeDtypeStruct((g2, n, n), jnp.float32),
        interpret=interpret,
    )(x, w, dinvb)


def _spread(ab, u, dinvb, act, interpret=False):
    g2, n, _ = ab.shape
    bm = 256

    def body(a_ref, uf_ref, dv_ref, o_ref):
        i = pl.program_id(1)
        a = a_ref[0].astype(jnp.float32)
        y = jnp.dot(a, uf_ref[0], preferred_element_type=jnp.float32)
        ue = uf_ref[0, pl.ds(i * bm, bm), :]
        y = (y + ue) * dv_ref[0, :, 0:1]
        if act:
            y = jnp.where(y >= 0, y, NEG_SLOPE * y)
        o_ref[0] = y

    return pl.pallas_call(
        body,
        grid=(g2, n // bm),
        in_specs=[
            pl.BlockSpec((1, bm, n), lambda g, i: (g, i, 0)),
            pl.BlockSpec((1, n, n), lambda g, i: (g, 0, 0)),
            pl.BlockSpec((1, bm, 128), lambda g, i: (g, i, 0)),
        ],
        out_specs=pl.BlockSpec((1, bm, n), lambda g, i: (g, i, 0)),
        out_shape=jax.ShapeDtypeStruct((g2, n, n), jnp.float32),
        interpret=interpret,
    )(ab, u, dinvb)


def _cross(x, interpret=False):
    _, n, _ = x.shape
    bm = 512

    def body(x1_ref, x2_ref, o_ref):
        o_ref[...] = lax.dot_general(
            x1_ref[0], x2_ref[0], (((0,), (0,)), ((), ())),
            preferred_element_type=jnp.float32)

    return pl.pallas_call(
        body,
        grid=(n // bm,),
        in_specs=[
            pl.BlockSpec((1, n, bm), lambda i: (0, 0, i)),
            pl.BlockSpec((1, n, n), lambda i: (1, 0, 0)),
        ],
        out_specs=pl.BlockSpec((bm, n), lambda i: (i, 0)),
        out_shape=jax.ShapeDtypeStruct((n, n), jnp.float32),
        interpret=interpret,
    )(x, x)


def kernel(edge_index_1, edge_index_2, feature, W):
    n = feature.shape[0]
    t1 = _matmul_plain(feature, W)
    a3 = _build_counts(edge_index_1, edge_index_2, n)
    dinvb, ab, wb = _rsqrt_deg(a3, W)
    x = _layer(t1, None, ab, dinvb, act=True, first=True)
    for layer in range(1, 4):
        x = _layer(x, wb, ab, dinvb, act=True, first=False)
    x = _layer(x, wb, ab, dinvb, act=False, first=False,
               out_dtype=jnp.float32)
    pred = _cross(x)
    return x[0], x[1], pred


# speedup vs baseline: 1.1223x; 1.0854x over previous
import jax
import jax.numpy as jnp
from jax import lax
from jax.experimental import pallas as pl
from jax.experimental.pallas import tpu as pltpu
from jax.experimental.pallas import tpu_sc as plsc

NEG_SLOPE = 0.01

_ROWS = 32
_NSUB = 16
_EDGE_BLK = 8192


def _build_counts(e1, e2, n, interpret=False):
    e = e1.shape[1]
    chunks = n // (_NSUB * _ROWS)
    eb_sz = min(e, _EDGE_BLK)
    n_eb = e // eb_sz
    unroll = 8

    def body(e1_hbm, e2_hbm, out_hbm, ebuf0, ebuf1, acc, sem0, sem1):
        g = lax.axis_index("c")
        s = lax.axis_index("s")

        def pipeline(ehbm):
            bufs = (ebuf0, ebuf1)
            sems = (sem0, sem1)
            for chunk in range(chunks):
                lo = (s * chunks + chunk) * _ROWS

                cps = {0: pltpu.async_copy(
                    ehbm.at[:, pl.ds(0, eb_sz)], bufs[0], sems[0])}

                zf = jnp.zeros((16,), jnp.float32)

                def zrow(r, carry):
                    @plsc.parallel_loop(0, n, 16, unroll=8)
                    def _(c):
                        acc[r, pl.ds(c, 16)] = zf
                    return carry

                lax.fori_loop(0, _ROWS, zrow, 0)

                lov = jnp.full((16,), lo, jnp.int32)
                rows_u = jnp.full((16,), _ROWS, jnp.uint32)

                for b in range(n_eb):
                    if b + 1 < n_eb:
                        cps[b + 1] = pltpu.async_copy(
                            ehbm.at[:, pl.ds((b + 1) * eb_sz, eb_sz)],
                            bufs[(b + 1) % 2], sems[(b + 1) % 2])
                    cps[b].wait()
                    buf = bufs[b % 2]

                    @plsc.parallel_loop(0, eb_sz, 16, unroll=unroll)
                    def _(off):
                        s16 = buf[0, pl.ds(off, 16)]
                        d16 = buf[1, pl.ds(off, 16)]
                        r16 = d16 - lov
                        m = plsc.bitcast(r16, jnp.uint32) < rows_u
                        plsc.addupdate_scatter(
                            acc, [r16, s16], m.astype(jnp.float32),
                            mask=m)

                pltpu.sync_copy(acc, out_hbm.at[g, pl.ds(lo, _ROWS), :])

        @pl.when(g == 0)
        def _():
            pipeline(e1_hbm)

        @pl.when(g == 1)
        def _():
            pipeline(e2_hbm)

    mesh = plsc.VectorSubcoreMesh(
        core_axis_name="c", subcore_axis_name="s",
        num_cores=2, num_subcores=_NSUB)
    f = pl.kernel(
        body,
        out_type=jax.ShapeDtypeStruct((2, n, n), jnp.float32),
        mesh=mesh,
        scratch_types=[
            pltpu.VMEM((2, eb_sz), jnp.int32),
            pltpu.VMEM((2, eb_sz), jnp.int32),
            pltpu.VMEM((_ROWS, n), jnp.float32),
            pltpu.SemaphoreType.DMA,
            pltpu.SemaphoreType.DMA,
        ],
        compiler_params=pltpu.CompilerParams(needs_layout_passes=False),
        interpret=interpret,
    )
    return f(e1, e2)


def _matmul_plain(x, w, interpret=False):
    n = x.shape[0]
    bm = 256

    def body(x_ref, w_ref, o_ref):
        o_ref[...] = jnp.dot(x_ref[...], w_ref[...],
                             preferred_element_type=jnp.float32
                             ).astype(jnp.bfloat16)

    return pl.pallas_call(
        body,
        grid=(n // bm,),
        in_specs=[
            pl.BlockSpec((bm, n), lambda i: (i, 0)),
            pl.BlockSpec((n, n), lambda i: (0, 0)),
        ],
        out_specs=pl.BlockSpec((bm, n), lambda i: (i, 0)),
        out_shape=jax.ShapeDtypeStruct((n, n), jnp.bfloat16),
        interpret=interpret,
    )(x, w)


def _scale_rows(t, dinvb, interpret=False):
    n = t.shape[0]
    g2 = dinvb.shape[0]
    bm = 256

    def body(t_ref, dv_ref, o_ref):
        o_ref[0] = t_ref[...] * dv_ref[0, :, 0:1]

    return pl.pallas_call(
        body,
        grid=(g2, n // bm),
        in_specs=[
            pl.BlockSpec((bm, n), lambda g, i: (i, 0)),
            pl.BlockSpec((1, bm, 128), lambda g, i: (g, i, 0)),
        ],
        out_specs=pl.BlockSpec((1, bm, n), lambda g, i: (g, i, 0)),
        out_shape=jax.ShapeDtypeStruct((g2, n, n), jnp.float32),
        interpret=interpret,
    )(t, dinvb)


def _rsqrt_deg(a3, w, interpret=False):
    g2, n, _ = a3.shape
    bm = 256

    def body(a_ref, w_ref, o_ref, ab_ref, wb_ref):
        a = a_ref[0]
        deg = jnp.sum(a, axis=1, keepdims=True) + 1.0
        o_ref[0] = jnp.broadcast_to(lax.rsqrt(deg), (a.shape[0], 128))
        ab_ref[0] = a.astype(jnp.bfloat16)
        wb_ref[...] = w_ref[...].astype(jnp.bfloat16)

    return pl.pallas_call(
        body,
        grid=(g2, n // bm),
        in_specs=[
            pl.BlockSpec((1, bm, n), lambda g, i: (g, i, 0)),
            pl.BlockSpec((bm, n), lambda g, i: (i, 0)),
        ],
        out_specs=[
            pl.BlockSpec((1, bm, 128), lambda g, i: (g, i, 0)),
            pl.BlockSpec((1, bm, n), lambda g, i: (g, i, 0)),
            pl.BlockSpec((bm, n), lambda g, i: (i, 0)),
        ],
        out_shape=[
            jax.ShapeDtypeStruct((g2, n, 128), jnp.float32),
            jax.ShapeDtypeStruct((g2, n, n), jnp.bfloat16),
            jax.ShapeDtypeStruct((n, n), jnp.bfloat16),
        ],
        interpret=interpret,
    )(a3, w)


def _layer(x, w, ab, dinvb, act, first, out_dtype=jnp.bfloat16,
           interpret=False):
    g2, n, _ = ab.shape
    bm = 256
    nb = n // bm

    def body(*refs):
        if first:
            x_ref, a_ref, dv_ref, o_ref, u_ref = refs
        else:
            x_ref, w_ref, a_ref, dv_ref, o_ref, u_ref = refs
        ph = pl.program_id(1)
        i = pl.program_id(2)
        dv = dv_ref[0, :, 0:1]

        @pl.when(ph == 0)
        def _():
            if first:
                t = x_ref[...].astype(jnp.float32)
            else:
                t = jnp.dot(x_ref[0], w_ref[...],
                            preferred_element_type=jnp.float32)
            u_ref[pl.ds(i * bm, bm), :] = (t * dv).astype(jnp.bfloat16)

        @pl.when(ph == 1)
        def _():
            y = jnp.dot(a_ref[0], u_ref[...],
                        preferred_element_type=jnp.float32)
            ue = u_ref[pl.ds(i * bm, bm), :].astype(jnp.float32)
            y = (y + ue) * dv
            if act:
                y = jnp.where(y >= 0, y, NEG_SLOPE * y)
            o_ref[0] = y.astype(out_dtype)

    if first:
        x_spec = pl.BlockSpec(
            (bm, n), lambda g, p, i: ((1 - p) * i + p * (nb - 1), 0))
        in_specs = [x_spec]
        args = (x,)
    else:
        x_spec = pl.BlockSpec(
            (1, bm, n), lambda g, p, i: (g, (1 - p) * i + p * (nb - 1), 0))
        w_spec = pl.BlockSpec((n, n), lambda g, p, i: (0, 0))
        in_specs = [x_spec, w_spec]
        args = (x, w)
    in_specs += [
        pl.BlockSpec((1, bm, n), lambda g, p, i: (g, p * i, 0)),
        pl.BlockSpec((1, bm, 128), lambda g, p, i: (g, i, 0)),
    ]
    args += (ab, dinvb)

    return pl.pallas_call(
        body,
        grid=(g2, 2, nb),
        in_specs=in_specs,
        out_specs=pl.BlockSpec((1, bm, n), lambda g, p, i: (g, p * i, 0)),
        out_shape=jax.ShapeDtypeStruct((g2, n, n), out_dtype),
        scratch_shapes=[pltpu.VMEM((n, n), jnp.bfloat16)],
        interpret=interpret,
    )(*args)


def _xw_scaled(x, w, dinvb, interpret=False):
    gx, n, _ = x.shape
    g2 = dinvb.shape[0]
    bm = 256

    def body(x_ref, w_ref, dv_ref, o_ref):
        xb = x_ref[0]
        dv = dv_ref[0, :, 0:1]
        o_ref[0] = jnp.dot(xb, w_ref[...], preferred_element_type=jnp.float32) * dv

    xmap = (lambda g, i: (g, i, 0)) if gx == g2 else (lambda g, i: (0, i, 0))
    return pl.pallas_call(
        body,
        grid=(g2, n // bm),
        in_specs=[
            pl.BlockSpec((1, bm, n), xmap),
            pl.BlockSpec((n, n), lambda g, i: (0, 0)),
            pl.BlockSpec((1, bm, 128), lambda g, i: (g, i, 0)),
        ],
        out_specs=pl.BlockSpec((1, bm, n), lambda g, i: (g, i, 0)),
        out_shape=jax.ShapeDtypeStruct((g2, n, n), jnp.float32),
        interpret=interpret,
    )(x, w, dinvb)


def _spread(ab, u, dinvb, act, interpret=False):
    g2, n, _ = ab.shape
    bm = 256

    def body(a_ref, uf_ref, dv_ref, o_ref):
        i = pl.program_id(1)
        a = a_ref[0].astype(jnp.float32)
        y = jnp.dot(a, uf_ref[0], preferred_element_type=jnp.float32)
        ue = uf_ref[0, pl.ds(i * bm, bm), :]
        y = (y + ue) * dv_ref[0, :, 0:1]
        if act:
            y = jnp.where(y >= 0, y, NEG_SLOPE * y)
        o_ref[0] = y

    return pl.pallas_call(
        body,
        grid=(g2, n // bm),
        in_specs=[
            pl.BlockSpec((1, bm, n), lambda g, i: (g, i, 0)),
            pl.BlockSpec((1, n, n), lambda g, i: (g, 0, 0)),
            pl.BlockSpec((1, bm, 128), lambda g, i: (g, i, 0)),
        ],
        out_specs=pl.BlockSpec((1, bm, n), lambda g, i: (g, i, 0)),
        out_shape=jax.ShapeDtypeStruct((g2, n, n), jnp.float32),
        interpret=interpret,
    )(ab, u, dinvb)


def _cross(x, interpret=False):
    _, n, _ = x.shape
    bm = 512

    def body(x1_ref, x2_ref, o_ref):
        o_ref[...] = lax.dot_general(
            x1_ref[0], x2_ref[0], (((0,), (0,)), ((), ())),
            preferred_element_type=jnp.float32)

    return pl.pallas_call(
        body,
        grid=(n // bm,),
        in_specs=[
            pl.BlockSpec((1, n, bm), lambda i: (0, 0, i)),
            pl.BlockSpec((1, n, n), lambda i: (1, 0, 0)),
        ],
        out_specs=pl.BlockSpec((bm, n), lambda i: (i, 0)),
        out_shape=jax.ShapeDtypeStruct((n, n), jnp.float32),
        interpret=interpret,
    )(x, x)


def kernel(edge_index_1, edge_index_2, feature, W):
    n = feature.shape[0]
    t1 = _matmul_plain(feature, W)
    a3 = _build_counts(edge_index_1, edge_index_2, n)
    dinvb, ab, wb = _rsqrt_deg(a3, W)
    x = _layer(t1, None, ab, dinvb, act=True, first=True)
    for layer in range(1, 4):
        x = _layer(x, wb, ab, dinvb, act=True, first=False)
    x = _layer(x, wb, ab, dinvb, act=False, first=False,
               out_dtype=jnp.float32)
    pred = _cross(x)
    return x[0], x[1], pred
